# baseline (speedup 1.0000x reference)
import functools

import jax
import jax.numpy as jnp
from jax import lax
from jax.experimental import pallas as pl
from jax.experimental.pallas import tpu as pltpu
from jax.experimental.pallas import tpu_sc as plsc

TOP_K = 8
D_MODEL = 4096
N_EXPERTS = 64
N_TOKENS = 32768

_CHUNKS = 8
_CT = N_TOKENS // _CHUNKS


_BT = 512


def _router_body(x_ref, w_ref, b_ref, logits_ref, probs_ref):
    l = jnp.dot(x_ref[...], w_ref[...], preferred_element_type=jnp.float32)
    l = l + b_ref[...]
    logits_ref[...] = l
    m = jnp.max(l, axis=1, keepdims=True)
    e = jnp.exp(l - m)
    s = jnp.sum(e, axis=1, keepdims=True)
    probs_ref[...] = (e / s).T


def _make_router(chunk):
    base = chunk * (_CT // _BT)
    return pl.pallas_call(
        _router_body,
        grid=(_CT // _BT,),
        in_specs=[
            pl.BlockSpec((_BT, D_MODEL), lambda i: (base + i, 0)),
            pl.BlockSpec((D_MODEL, N_EXPERTS), lambda i: (0, 0)),
            pl.BlockSpec((1, N_EXPERTS), lambda i: (0, 0)),
        ],
        out_specs=[
            pl.BlockSpec((_BT, N_EXPERTS), lambda i: (i, 0)),
            pl.BlockSpec((N_EXPERTS, _BT), lambda i: (0, i)),
        ],
        out_shape=[
            jax.ShapeDtypeStruct((_CT, N_EXPERTS), jnp.float32),
            jax.ShapeDtypeStruct((N_EXPERTS, _CT), jnp.float32),
        ],
        compiler_params=pltpu.CompilerParams(
            dimension_semantics=("parallel",),
        ),
    )


_routers = [_make_router(c) for c in range(_CHUNKS)]


_NC = 2
_NS = 16
_NW = _NC * _NS
_L = 16
_TPW = _CT // _NW
_GROUPS = _TPW // _L


_SORT8 = [
    (0, 1), (2, 3), (4, 5), (6, 7),
    (0, 2), (1, 3), (4, 6), (5, 7),
    (1, 2), (5, 6),
    (0, 4), (1, 5), (2, 6), (3, 7),
    (2, 4), (3, 5),
    (1, 2), (3, 4), (5, 6),
]
_BITONIC8 = [
    (0, 4), (1, 5), (2, 6), (3, 7),
    (0, 2), (1, 3), (4, 6), (5, 7),
    (0, 1), (2, 3), (4, 5), (6, 7),
]


def _ce(vals, idxs, i, j):
    c = vals[i] < vals[j]
    vals[i], vals[j] = (jnp.where(c, vals[j], vals[i]),
                        jnp.where(c, vals[i], vals[j]))
    idxs[i], idxs[j] = (jnp.where(c, idxs[j], idxs[i]),
                        jnp.where(c, idxs[i], idxs[j]))


def _merge_top8(a_v, a_i, b_v, b_i):
    vals, idxs = [], []
    for k in range(TOP_K):
        c = a_v[k] >= b_v[TOP_K - 1 - k]
        vals.append(jnp.where(c, a_v[k], b_v[TOP_K - 1 - k]))
        idxs.append(jnp.where(c, a_i[k], b_i[TOP_K - 1 - k]))
    for i, j in _BITONIC8:
        _ce(vals, idxs, i, j)
    return vals, idxs


def _topk_body(probs_hbm, w_out, i_out, p_v, w_v, i_v):
    wid = lax.axis_index("s") * _NC + lax.axis_index("c")
    base = wid * _TPW
    pltpu.sync_copy(probs_hbm.at[:, pl.ds(base, _TPW)], p_v)

    def group(g, carry):
        tok = g * _L + lax.iota(jnp.int32, _L)
        lists = []
        for grp in range(N_EXPERTS // TOP_K):
            vals = []
            idxs = []
            for k in range(TOP_K):
                e = grp * TOP_K + k
                vals.append(p_v[e, pl.ds(g * _L, _L)])
                idxs.append(jnp.full((_L,), e, jnp.int32))
            for i, j in _SORT8:
                _ce(vals, idxs, i, j)
            lists.append((vals, idxs))
        while len(lists) > 1:
            nxt = []
            for p in range(0, len(lists), 2):
                av, ai = lists[p]
                bv, bi = lists[p + 1]
                nxt.append(_merge_top8(av, ai, bv, bi))
            lists = nxt
        vals, idxs = lists[0]
        for j in range(TOP_K):
            jx = jnp.full((_L,), j, jnp.int32)
            plsc.store_scatter(w_v, [tok, jx], vals[j])
            plsc.store_scatter(i_v, [tok, jx], idxs[j])
        return carry

    lax.fori_loop(0, _GROUPS, group, 0)
    pltpu.sync_copy(w_v, w_out.at[pl.ds(base, _TPW)])
    pltpu.sync_copy(i_v, i_out.at[pl.ds(base, _TPW)])


_sc_topk = functools.partial(
    pl.kernel,
    out_type=[
        jax.ShapeDtypeStruct((_CT, TOP_K), jnp.float32),
        jax.ShapeDtypeStruct((_CT, TOP_K), jnp.int32),
    ],
    mesh=plsc.VectorSubcoreMesh(core_axis_name="c", subcore_axis_name="s"),
    scratch_types=[
        pltpu.VMEM((N_EXPERTS, _TPW), jnp.float32),
        pltpu.VMEM((_TPW, TOP_K), jnp.float32),
        pltpu.VMEM((_TPW, TOP_K), jnp.int32),
    ],
    compiler_params=pltpu.CompilerParams(needs_layout_passes=False),
)(_topk_body)


def kernel(x, W, b):
    b2 = b.reshape(1, N_EXPERTS)
    logits_c, w_c, i_c = [], [], []
    for c in range(_CHUNKS):
        logits, probs = _routers[c](x, W, b2)
        tw, ti = _sc_topk(probs)
        logits_c.append(logits)
        w_c.append(tw)
        i_c.append(ti)
    return (
        jnp.concatenate(logits_c, axis=0),
        jnp.concatenate(w_c, axis=0),
        jnp.concatenate(i_c, axis=0),
    )


# speedup vs baseline: 1.1248x; 1.1248x over previous
import functools

import jax
import jax.numpy as jnp
from jax import lax
from jax.experimental import pallas as pl
from jax.experimental.pallas import tpu as pltpu
from jax.experimental.pallas import tpu_sc as plsc

TOP_K = 8
D_MODEL = 4096
N_EXPERTS = 64
N_TOKENS = 32768

_CHUNKS = 4
_CT = N_TOKENS // _CHUNKS


_BT = 1024


def _router_body(x_ref, w_ref, b_ref, logits_ref, probs_ref):
    l = jnp.dot(x_ref[...], w_ref[...], preferred_element_type=jnp.float32)
    l = l + b_ref[...]
    logits_ref[...] = l
    m = jnp.max(l, axis=1, keepdims=True)
    e = jnp.exp(l - m)
    s = jnp.sum(e, axis=1, keepdims=True)
    probs_ref[...] = (e / s).T


def _make_router(chunk):
    base = chunk * (_CT // _BT)
    return pl.pallas_call(
        _router_body,
        grid=(_CT // _BT,),
        in_specs=[
            pl.BlockSpec((_BT, D_MODEL), lambda i: (base + i, 0)),
            pl.BlockSpec((D_MODEL, N_EXPERTS), lambda i: (0, 0)),
            pl.BlockSpec((1, N_EXPERTS), lambda i: (0, 0)),
        ],
        out_specs=[
            pl.BlockSpec((_BT, N_EXPERTS), lambda i: (i, 0)),
            pl.BlockSpec((N_EXPERTS, _BT), lambda i: (0, i)),
        ],
        out_shape=[
            jax.ShapeDtypeStruct((_CT, N_EXPERTS), jnp.float32),
            jax.ShapeDtypeStruct((N_EXPERTS, _CT), jnp.float32),
        ],
        compiler_params=pltpu.CompilerParams(
            dimension_semantics=("parallel",),
        ),
    )


_routers = [_make_router(c) for c in range(_CHUNKS)]


_NC = 2
_NS = 16
_NW = _NC * _NS
_L = 16
_TPW = _CT // _NW
_GROUPS = _TPW // _L


_SORT8 = [
    (0, 1), (2, 3), (4, 5), (6, 7),
    (0, 2), (1, 3), (4, 6), (5, 7),
    (1, 2), (5, 6),
    (0, 4), (1, 5), (2, 6), (3, 7),
    (2, 4), (3, 5),
    (1, 2), (3, 4), (5, 6),
]
_BITONIC8 = [
    (0, 4), (1, 5), (2, 6), (3, 7),
    (0, 2), (1, 3), (4, 6), (5, 7),
    (0, 1), (2, 3), (4, 5), (6, 7),
]


def _ce(vals, idxs, i, j):
    c = vals[i] < vals[j]
    vals[i], vals[j] = (jnp.where(c, vals[j], vals[i]),
                        jnp.where(c, vals[i], vals[j]))
    idxs[i], idxs[j] = (jnp.where(c, idxs[j], idxs[i]),
                        jnp.where(c, idxs[i], idxs[j]))


def _merge_top8(a_v, a_i, b_v, b_i):
    vals, idxs = [], []
    for k in range(TOP_K):
        c = a_v[k] >= b_v[TOP_K - 1 - k]
        vals.append(jnp.where(c, a_v[k], b_v[TOP_K - 1 - k]))
        idxs.append(jnp.where(c, a_i[k], b_i[TOP_K - 1 - k]))
    for i, j in _BITONIC8:
        _ce(vals, idxs, i, j)
    return vals, idxs


def _topk_body(probs_hbm, w_out, i_out, p_v, w_v, i_v):
    wid = lax.axis_index("s") * _NC + lax.axis_index("c")
    base = wid * _TPW
    pltpu.sync_copy(probs_hbm.at[:, pl.ds(base, _TPW)], p_v)

    def group(g, carry):
        tok = g * _L + lax.iota(jnp.int32, _L)
        lists = []
        for grp in range(N_EXPERTS // TOP_K):
            vals = []
            idxs = []
            for k in range(TOP_K):
                e = grp * TOP_K + k
                vals.append(p_v[e, pl.ds(g * _L, _L)])
                idxs.append(jnp.full((_L,), e, jnp.int32))
            for i, j in _SORT8:
                _ce(vals, idxs, i, j)
            lists.append((vals, idxs))
        while len(lists) > 1:
            nxt = []
            for p in range(0, len(lists), 2):
                av, ai = lists[p]
                bv, bi = lists[p + 1]
                nxt.append(_merge_top8(av, ai, bv, bi))
            lists = nxt
        vals, idxs = lists[0]
        for j in range(TOP_K):
            jx = jnp.full((_L,), j, jnp.int32)
            plsc.store_scatter(w_v, [tok, jx], vals[j])
            plsc.store_scatter(i_v, [tok, jx], idxs[j])
        return carry

    lax.fori_loop(0, _GROUPS, group, 0)
    pltpu.sync_copy(w_v, w_out.at[pl.ds(base, _TPW)])
    pltpu.sync_copy(i_v, i_out.at[pl.ds(base, _TPW)])


_sc_topk = functools.partial(
    pl.kernel,
    out_type=[
        jax.ShapeDtypeStruct((_CT, TOP_K), jnp.float32),
        jax.ShapeDtypeStruct((_CT, TOP_K), jnp.int32),
    ],
    mesh=plsc.VectorSubcoreMesh(core_axis_name="c", subcore_axis_name="s"),
    scratch_types=[
        pltpu.VMEM((N_EXPERTS, _TPW), jnp.float32),
        pltpu.VMEM((_TPW, TOP_K), jnp.float32),
        pltpu.VMEM((_TPW, TOP_K), jnp.int32),
    ],
    compiler_params=pltpu.CompilerParams(needs_layout_passes=False),
)(_topk_body)


def kernel(x, W, b):
    b2 = b.reshape(1, N_EXPERTS)
    logits_c, w_c, i_c = [], [], []
    for c in range(_CHUNKS):
        logits, probs = _routers[c](x, W, b2)
        tw, ti = _sc_topk(probs)
        logits_c.append(logits)
        w_c.append(tw)
        i_c.append(ti)
    return (
        jnp.concatenate(logits_c, axis=0),
        jnp.concatenate(w_c, axis=0),
        jnp.concatenate(i_c, axis=0),
    )
